# baseline (speedup 1.0000x reference)
import dataclasses
import functools

import jax
import jax.numpy as jnp
from jax import lax
from jax.experimental import pallas as pl
from jax.experimental.pallas import tpu as pltpu
from jax.experimental.pallas import tpu_sc as plsc

N_NODES = 10000
N_EDGES = 320000
IN_CH = 128
HID_CH = 64

NC = 2
NS = 16
NW = NC * NS
L = 16

CHUNK = 128
PT_CH = 80
PT1_CH = 160
NBUF = 4
SC_CORE = 1
PT_E = PT_CH * CHUNK
E_PAD = NW * PT_E
PAD_N = 10240
RPT = PAD_N // NS
RB = 512
CB = 2048


def _mesh():
    return plsc.VectorSubcoreMesh(core_axis_name="c", subcore_axis_name="s")


def _sc_params():
    cp = pltpu.CompilerParams()
    fields = pltpu.CompilerParams.__dataclass_fields__
    if "needs_layout_passes" in fields:
        cp = dataclasses.replace(cp, needs_layout_passes=False)
    if "use_tc_tiling_on_sc" in fields:
        cp = dataclasses.replace(cp, use_tc_tiling_on_sc=False)
    return cp


def _deg_partials(dst3):
    @functools.partial(
        pl.kernel,
        out_type=jax.ShapeDtypeStruct((NW, PAD_N), jnp.float32),
        mesh=_mesh(),
        compiler_params=_sc_params(),
        scratch_types=[
            pltpu.VMEM((PT_CH, CHUNK), jnp.int32),
            pltpu.VMEM((PAD_N,), jnp.float32),
        ],
    )
    def k(dst_hbm, out_hbm, dst_v, acc_v):
        wid = lax.axis_index("s") * NC + lax.axis_index("c")
        pltpu.sync_copy(dst_hbm.at[wid], dst_v)
        zeros = jnp.zeros((L,), jnp.float32)
        ones = jnp.ones((L,), jnp.float32)

        @pl.loop(0, PAD_N, step=L)
        def _(i):
            acc_v[pl.ds(i, L)] = zeros

        @pl.loop(0, PT_CH)
        def _(c):
            @pl.loop(0, CHUNK, step=L)
            def _(j):
                idx = dst_v[c, pl.ds(j, L)]
                plsc.addupdate_scatter(acc_v, [idx], ones)

        pltpu.sync_copy(acc_v, out_hbm.at[wid])

    return k(dst3)


def _agg1_partials(hs, src3, dst3):
    @functools.partial(
        pl.kernel,
        out_type=jax.ShapeDtypeStruct((NC, PAD_N, HID_CH), jnp.float32),
        mesh=_mesh(),
        compiler_params=_sc_params(),
        scratch_types=[
            pltpu.VMEM((PT_CH, CHUNK), jnp.int32),
            pltpu.VMEM((PT_CH, CHUNK), jnp.int32),
            pltpu.VMEM((CHUNK, HID_CH), jnp.float32),
            pltpu.VMEM_SHARED((PAD_N, HID_CH), jnp.float32),
            pltpu.SemaphoreType.DMA,
        ],
    )
    def k(hs_hbm, src_hbm, dst_hbm, out_hbm, src_v, dst_v, rows_v, acc_sh, sem):
        cid = lax.axis_index("c")
        sid = lax.axis_index("s")
        wid = sid * NC + cid
        pltpu.sync_copy(src_hbm.at[wid], src_v)
        pltpu.sync_copy(dst_hbm.at[wid], dst_v)

        zeros = jnp.zeros((L,), jnp.float32)

        @pl.loop(0, CHUNK)
        def _(r):
            @pl.loop(0, HID_CH, step=L)
            def _(c):
                rows_v[r, pl.ds(c, L)] = zeros

        @pl.loop(0, RPT, step=CHUNK)
        def _(r0):
            pltpu.sync_copy(rows_v, acc_sh.at[pl.ds(sid * RPT + r0, CHUNK)])

        plsc.subcore_barrier()

        @pl.loop(0, PT_CH)
        def _(c):
            pltpu.async_copy(hs_hbm.at[src_v.at[c]], rows_v, sem).wait()
            pltpu.sync_copy(rows_v, acc_sh.at[dst_v.at[c]], add=True)

        plsc.subcore_barrier()

        pltpu.sync_copy(
            acc_sh.at[pl.ds(sid * RPT, RPT)],
            out_hbm.at[cid, pl.ds(sid * RPT, RPT)],
        )

    return k(hs, src3, dst3)


def _agg2_partials(gs, src3, dst3):
    @functools.partial(
        pl.kernel,
        out_type=jax.ShapeDtypeStruct((NW, PAD_N), jnp.float32),
        mesh=_mesh(),
        compiler_params=_sc_params(),
        scratch_types=[
            pltpu.VMEM((PT_CH, CHUNK), jnp.int32),
            pltpu.VMEM((PT_CH, CHUNK), jnp.int32),
            pltpu.VMEM((PAD_N,), jnp.float32),
            pltpu.VMEM((PAD_N,), jnp.float32),
        ],
    )
    def k(gs_hbm, src_hbm, dst_hbm, out_hbm, src_v, dst_v, tab_v, acc_v):
        wid = lax.axis_index("s") * NC + lax.axis_index("c")
        pltpu.sync_copy(src_hbm.at[wid], src_v)
        pltpu.sync_copy(dst_hbm.at[wid], dst_v)
        pltpu.sync_copy(gs_hbm, tab_v)
        zeros = jnp.zeros((L,), jnp.float32)

        @pl.loop(0, PAD_N, step=L)
        def _(i):
            acc_v[pl.ds(i, L)] = zeros

        @pl.loop(0, PT_CH)
        def _(c):
            @pl.loop(0, CHUNK, step=L)
            def _(j):
                si = src_v[c, pl.ds(j, L)]
                di = dst_v[c, pl.ds(j, L)]
                vals = plsc.load_gather(tab_v, [si])
                plsc.addupdate_scatter(acc_v, [di], vals)

        pltpu.sync_copy(acc_v, out_hbm.at[wid])

    return k(gs, src3, dst3)


def _tc_scale_matmul(deg_part, x_pad, W1):
    def body(dp_ref, x_ref, w1_ref, hs_ref, dis_ref):
        i = pl.program_id(0)
        deg = jnp.sum(dp_ref[...], axis=0)[:, None] + 1.0
        disv = lax.rsqrt(deg)
        rid = i * RB + lax.broadcasted_iota(jnp.int32, (RB, 1), 0)
        disv = jnp.where(rid < N_NODES, disv, 0.0)
        dis_ref[...] = disv
        h = jnp.dot(x_ref[...], w1_ref[...], preferred_element_type=jnp.float32)
        hs_ref[...] = h * disv

    return pl.pallas_call(
        body,
        grid=(PAD_N // RB,),
        in_specs=[
            pl.BlockSpec((NW, RB), lambda i: (0, i)),
            pl.BlockSpec((RB, IN_CH), lambda i: (i, 0)),
            pl.BlockSpec((IN_CH, HID_CH), lambda i: (0, 0)),
        ],
        out_specs=[
            pl.BlockSpec((RB, HID_CH), lambda i: (i, 0)),
            pl.BlockSpec((RB, 1), lambda i: (i, 0)),
        ],
        out_shape=[
            jax.ShapeDtypeStruct((PAD_N, HID_CH), jnp.float32),
            jax.ShapeDtypeStruct((PAD_N, 1), jnp.float32),
        ],
    )(deg_part, x_pad, W1)


def _tc_layer2_scalar(acc1, hs, dis, b1_row, w2_row):
    def body(acc_ref, hs_ref, dis_ref, b1_ref, w2_ref, gs_ref):
        s = acc_ref[0] + acc_ref[1] + hs_ref[...]
        pre = s * dis_ref[...] + b1_ref[...]
        h2 = jnp.maximum(pre, 0.0)
        g = jnp.sum(h2 * w2_ref[...], axis=1, keepdims=True)
        gs_ref[...] = g * dis_ref[...]

    return pl.pallas_call(
        body,
        grid=(PAD_N // RB,),
        in_specs=[
            pl.BlockSpec((NC, RB, HID_CH), lambda i: (0, i, 0)),
            pl.BlockSpec((RB, HID_CH), lambda i: (i, 0)),
            pl.BlockSpec((RB, 1), lambda i: (i, 0)),
            pl.BlockSpec((1, HID_CH), lambda i: (0, 0)),
            pl.BlockSpec((1, HID_CH), lambda i: (0, 0)),
        ],
        out_specs=pl.BlockSpec((RB, 1), lambda i: (i, 0)),
        out_shape=jax.ShapeDtypeStruct((PAD_N, 1), jnp.float32),
    )(acc1, hs, dis, b1_row, w2_row)


def _tc_final(acc2, gs, dis, b2_11):
    def body(a2_ref, gs_ref, dis_ref, b2_ref, o_ref):
        a2 = jnp.sum(a2_ref[...], axis=0)[:, None]
        o_ref[...] = jax.nn.sigmoid(
            (a2 + gs_ref[...]) * dis_ref[...] + b2_ref[...]
        )

    return pl.pallas_call(
        body,
        grid=(PAD_N // CB,),
        in_specs=[
            pl.BlockSpec((NW, CB), lambda i: (0, i)),
            pl.BlockSpec((CB, 1), lambda i: (i, 0)),
            pl.BlockSpec((CB, 1), lambda i: (i, 0)),
            pl.BlockSpec((1, 1), lambda i: (0, 0)),
        ],
        out_specs=pl.BlockSpec((CB, 1), lambda i: (i, 0)),
        out_shape=jax.ShapeDtypeStruct((PAD_N, 1), jnp.float32),
    )(acc2, gs, dis, b2_11)


def kernel(x, edge_index, W1, b1, W2, b2):
    ei = edge_index.astype(jnp.int32)
    pad = jnp.full((E_PAD - N_EDGES,), N_NODES, jnp.int32)
    src_flat = jnp.concatenate([ei[0], pad])
    dst_flat = jnp.concatenate([ei[1], pad])
    src3 = src_flat.reshape(NW, PT_CH, CHUNK)
    dst3 = dst_flat.reshape(NW, PT_CH, CHUNK)
    x_pad = jnp.pad(x, ((0, PAD_N - N_NODES), (0, 0)))

    deg_part = _deg_partials(dst3)
    hs, dis = _tc_scale_matmul(deg_part, x_pad, W1)
    acc1 = _agg1_partials(hs, src3, dst3)
    gs = _tc_layer2_scalar(
        acc1, hs, dis, b1.reshape(1, HID_CH), W2.reshape(1, HID_CH)
    )
    acc2 = _agg2_partials(gs.reshape(PAD_N), src3, dst3)
    out = _tc_final(acc2, gs, dis, b2.reshape(1, 1))
    return out[:N_NODES]


# speedup vs baseline: 1.9695x; 1.9695x over previous
import dataclasses
import functools

import jax
import jax.numpy as jnp
from jax import lax
from jax.experimental import pallas as pl
from jax.experimental.pallas import tpu as pltpu
from jax.experimental.pallas import tpu_sc as plsc

N_NODES = 10000
N_EDGES = 320000
IN_CH = 128
HID_CH = 64

NC = 2
NS = 16
NW = NC * NS
L = 16

CHUNK = 128
PT_CH = 80
PT1_CH = 160
NBUF = 4
SC_CORE = 1
PT_E = PT_CH * CHUNK
E_PAD = NW * PT_E
PAD_N = 10240
RPT = PAD_N // NS
RB = 512
CB = 2048


def _mesh():
    return plsc.VectorSubcoreMesh(core_axis_name="c", subcore_axis_name="s")


def _sc_params():
    cp = pltpu.CompilerParams()
    fields = pltpu.CompilerParams.__dataclass_fields__
    if "needs_layout_passes" in fields:
        cp = dataclasses.replace(cp, needs_layout_passes=False)
    if "use_tc_tiling_on_sc" in fields:
        cp = dataclasses.replace(cp, use_tc_tiling_on_sc=False)
    return cp


def _deg_partials(dst3):
    @functools.partial(
        pl.kernel,
        out_type=jax.ShapeDtypeStruct((NW, PAD_N), jnp.float32),
        mesh=_mesh(),
        compiler_params=_sc_params(),
        scratch_types=[
            pltpu.VMEM((PT_CH, CHUNK), jnp.int32),
            pltpu.VMEM((PAD_N,), jnp.float32),
        ],
    )
    def k(dst_hbm, out_hbm, dst_v, acc_v):
        wid = lax.axis_index("s") * NC + lax.axis_index("c")
        pltpu.sync_copy(dst_hbm.at[wid], dst_v)
        zeros = jnp.zeros((L,), jnp.float32)
        ones = jnp.ones((L,), jnp.float32)

        @pl.loop(0, PAD_N, step=L)
        def _(i):
            acc_v[pl.ds(i, L)] = zeros

        @pl.loop(0, PT_CH)
        def _(c):
            @pl.loop(0, CHUNK, step=L)
            def _(j):
                idx = dst_v[c, pl.ds(j, L)]
                plsc.addupdate_scatter(acc_v, [idx], ones)

        pltpu.sync_copy(acc_v, out_hbm.at[wid])

    return k(dst3)


def _agg1_partials(hs, src3, dst3):
    @functools.partial(
        pl.kernel,
        out_type=jax.ShapeDtypeStruct((NC, PAD_N, HID_CH), jnp.float32),
        mesh=_mesh(),
        compiler_params=_sc_params(),
        scratch_types=[
            pltpu.VMEM((PT_CH, CHUNK), jnp.int32),
            pltpu.VMEM((PT_CH, CHUNK), jnp.int32),
            pltpu.VMEM((CHUNK, HID_CH), jnp.float32),
            pltpu.VMEM_SHARED((PAD_N, HID_CH), jnp.float32),
            pltpu.SemaphoreType.DMA,
        ],
    )
    def k(hs_hbm, src_hbm, dst_hbm, out_hbm, src_v, dst_v, rows_v, acc_sh, sem):
        cid = lax.axis_index("c")
        sid = lax.axis_index("s")
        wid = sid * NC + cid
        pltpu.sync_copy(src_hbm.at[wid], src_v)
        pltpu.sync_copy(dst_hbm.at[wid], dst_v)

        zeros = jnp.zeros((L,), jnp.float32)

        @pl.loop(0, CHUNK)
        def _(r):
            @pl.loop(0, HID_CH, step=L)
            def _(c):
                rows_v[r, pl.ds(c, L)] = zeros

        @pl.loop(0, RPT, step=CHUNK)
        def _(r0):
            pltpu.sync_copy(rows_v, acc_sh.at[pl.ds(sid * RPT + r0, CHUNK)])

        plsc.subcore_barrier()

        @pl.loop(0, PT_CH)
        def _(c):
            pltpu.async_copy(hs_hbm.at[src_v.at[c]], rows_v, sem).wait()
            pltpu.sync_copy(rows_v, acc_sh.at[dst_v.at[c]], add=True)

        plsc.subcore_barrier()

        pltpu.sync_copy(
            acc_sh.at[pl.ds(sid * RPT, RPT)],
            out_hbm.at[cid, pl.ds(sid * RPT, RPT)],
        )

    return k(hs, src3, dst3)


def _agg2_partials(gs, src3, dst3):
    @functools.partial(
        pl.kernel,
        out_type=jax.ShapeDtypeStruct((NW, PAD_N), jnp.float32),
        mesh=_mesh(),
        compiler_params=_sc_params(),
        scratch_types=[
            pltpu.VMEM((PT_CH, CHUNK), jnp.int32),
            pltpu.VMEM((PT_CH, CHUNK), jnp.int32),
            pltpu.VMEM((PAD_N,), jnp.float32),
            pltpu.VMEM((PAD_N,), jnp.float32),
        ],
    )
    def k(gs_hbm, src_hbm, dst_hbm, out_hbm, src_v, dst_v, tab_v, acc_v):
        wid = lax.axis_index("s") * NC + lax.axis_index("c")
        pltpu.sync_copy(src_hbm.at[wid], src_v)
        pltpu.sync_copy(dst_hbm.at[wid], dst_v)
        pltpu.sync_copy(gs_hbm, tab_v)
        zeros = jnp.zeros((L,), jnp.float32)

        @pl.loop(0, PAD_N, step=L)
        def _(i):
            acc_v[pl.ds(i, L)] = zeros

        @pl.loop(0, PT_CH)
        def _(c):
            @pl.loop(0, CHUNK, step=L)
            def _(j):
                si = src_v[c, pl.ds(j, L)]
                di = dst_v[c, pl.ds(j, L)]
                vals = plsc.load_gather(tab_v, [si])
                plsc.addupdate_scatter(acc_v, [di], vals)

        pltpu.sync_copy(acc_v, out_hbm.at[wid])

    return k(gs, src3, dst3)


def _tc_scale_matmul(deg_part, x_pad, W1):
    def body(dp_ref, x_ref, w1_ref, hs_ref, dis_ref):
        i = pl.program_id(0)
        deg = jnp.sum(dp_ref[...], axis=0)[:, None] + 1.0
        disv = lax.rsqrt(deg)
        rid = i * RB + lax.broadcasted_iota(jnp.int32, (RB, 1), 0)
        disv = jnp.where(rid < N_NODES, disv, 0.0)
        dis_ref[...] = disv
        h = jnp.dot(x_ref[...], w1_ref[...], preferred_element_type=jnp.float32)
        hs_ref[...] = h * disv

    return pl.pallas_call(
        body,
        grid=(PAD_N // RB,),
        in_specs=[
            pl.BlockSpec((NW, RB), lambda i: (0, i)),
            pl.BlockSpec((RB, IN_CH), lambda i: (i, 0)),
            pl.BlockSpec((IN_CH, HID_CH), lambda i: (0, 0)),
        ],
        out_specs=[
            pl.BlockSpec((RB, HID_CH), lambda i: (i, 0)),
            pl.BlockSpec((RB, 1), lambda i: (i, 0)),
        ],
        out_shape=[
            jax.ShapeDtypeStruct((PAD_N, HID_CH), jnp.float32),
            jax.ShapeDtypeStruct((PAD_N, 1), jnp.float32),
        ],
    )(deg_part, x_pad, W1)


def _tc_layer2_scalar(acc1, hs, dis, b1_row, w2_row):
    def body(acc_ref, hs_ref, dis_ref, b1_ref, w2_ref, gs_ref):
        s = acc_ref[0] + acc_ref[1] + hs_ref[...]
        pre = s * dis_ref[...] + b1_ref[...]
        h2 = jnp.maximum(pre, 0.0)
        g = jnp.sum(h2 * w2_ref[...], axis=1, keepdims=True)
        gs_ref[...] = g * dis_ref[...]

    return pl.pallas_call(
        body,
        grid=(PAD_N // RB,),
        in_specs=[
            pl.BlockSpec((NC, RB, HID_CH), lambda i: (0, i, 0)),
            pl.BlockSpec((RB, HID_CH), lambda i: (i, 0)),
            pl.BlockSpec((RB, 1), lambda i: (i, 0)),
            pl.BlockSpec((1, HID_CH), lambda i: (0, 0)),
            pl.BlockSpec((1, HID_CH), lambda i: (0, 0)),
        ],
        out_specs=pl.BlockSpec((RB, 1), lambda i: (i, 0)),
        out_shape=jax.ShapeDtypeStruct((PAD_N, 1), jnp.float32),
    )(acc1, hs, dis, b1_row, w2_row)


def _tc_final(acc2, gs, dis, b2_11):
    def body(a2_ref, gs_ref, dis_ref, b2_ref, o_ref):
        a2 = jnp.sum(a2_ref[...], axis=0)[:, None]
        o_ref[...] = jax.nn.sigmoid(
            (a2 + gs_ref[...]) * dis_ref[...] + b2_ref[...]
        )

    return pl.pallas_call(
        body,
        grid=(PAD_N // CB,),
        in_specs=[
            pl.BlockSpec((NW, CB), lambda i: (0, i)),
            pl.BlockSpec((CB, 1), lambda i: (i, 0)),
            pl.BlockSpec((CB, 1), lambda i: (i, 0)),
            pl.BlockSpec((1, 1), lambda i: (0, 0)),
        ],
        out_specs=pl.BlockSpec((CB, 1), lambda i: (i, 0)),
        out_shape=jax.ShapeDtypeStruct((PAD_N, 1), jnp.float32),
    )(acc2, gs, dis, b2_11)


def kernel(x, edge_index, W1, b1, W2, b2):
    ei = edge_index.astype(jnp.int32)
    pad = N_NODES + jnp.arange(E_PAD - N_EDGES, dtype=jnp.int32) % (PAD_N - N_NODES)
    src_flat = jnp.concatenate([ei[0], pad])
    dst_flat = jnp.concatenate([ei[1], pad])
    src3 = src_flat.reshape(NW, PT_CH, CHUNK)
    dst3 = dst_flat.reshape(NW, PT_CH, CHUNK)
    x_pad = jnp.pad(x, ((0, PAD_N - N_NODES), (0, 0)))

    deg_part = _deg_partials(dst3)
    hs, dis = _tc_scale_matmul(deg_part, x_pad, W1)
    acc1 = _agg1_partials(hs, src3, dst3)
    gs = _tc_layer2_scalar(
        acc1, hs, dis, b1.reshape(1, HID_CH), W2.reshape(1, HID_CH)
    )
    acc2 = _agg2_partials(gs.reshape(PAD_N), src3, dst3)
    out = _tc_final(acc2, gs, dis, b2.reshape(1, 1))
    return out[:N_NODES]


# speedup vs baseline: 2.6790x; 1.3603x over previous
import dataclasses
import functools

import jax
import jax.numpy as jnp
from jax import lax
from jax.experimental import pallas as pl
from jax.experimental.pallas import tpu as pltpu
from jax.experimental.pallas import tpu_sc as plsc

N_NODES = 10000
N_EDGES = 320000
IN_CH = 128
HID_CH = 64

NC = 2
NS = 16
NW = NC * NS
L = 16

CHUNK = 128
PT_CH = 80
PT1_CH = 160
NBUF = 4
SC_CORE = 1
PT_E = PT_CH * CHUNK
E_PAD = NW * PT_E
PAD_N = 10240
RPT = PAD_N // NS
RB = 512
CB = 2048


def _mesh():
    return plsc.VectorSubcoreMesh(core_axis_name="c", subcore_axis_name="s")


def _sc_params():
    cp = pltpu.CompilerParams()
    fields = pltpu.CompilerParams.__dataclass_fields__
    if "needs_layout_passes" in fields:
        cp = dataclasses.replace(cp, needs_layout_passes=False)
    if "use_tc_tiling_on_sc" in fields:
        cp = dataclasses.replace(cp, use_tc_tiling_on_sc=False)
    return cp


def _deg_partials(dst3):
    @functools.partial(
        pl.kernel,
        out_type=jax.ShapeDtypeStruct((NW, PAD_N), jnp.float32),
        mesh=_mesh(),
        compiler_params=_sc_params(),
        scratch_types=[
            pltpu.VMEM((PT_CH, CHUNK), jnp.int32),
            pltpu.VMEM((PAD_N,), jnp.float32),
        ],
    )
    def k(dst_hbm, out_hbm, dst_v, acc_v):
        wid = lax.axis_index("s") * NC + lax.axis_index("c")
        pltpu.sync_copy(dst_hbm.at[wid], dst_v)
        zeros = jnp.zeros((L,), jnp.float32)
        ones = jnp.ones((L,), jnp.float32)

        @pl.loop(0, PAD_N, step=L)
        def _(i):
            acc_v[pl.ds(i, L)] = zeros

        @pl.loop(0, PT_CH)
        def _(c):
            @pl.loop(0, CHUNK, step=L)
            def _(j):
                idx = dst_v[c, pl.ds(j, L)]
                plsc.addupdate_scatter(acc_v, [idx], ones)

        pltpu.sync_copy(acc_v, out_hbm.at[wid])

    return k(dst3)


def _agg1_partials(hs, src3, dst3):
    @functools.partial(
        pl.kernel,
        out_type=jax.ShapeDtypeStruct((NC, PAD_N, HID_CH), jnp.float32),
        mesh=_mesh(),
        compiler_params=_sc_params(),
        scratch_types=[
            pltpu.VMEM((PT_CH, CHUNK), jnp.int32),
            pltpu.VMEM((PT_CH, CHUNK), jnp.int32),
            pltpu.VMEM((NBUF, CHUNK, HID_CH), jnp.float32),
            pltpu.VMEM_SHARED((PAD_N, HID_CH), jnp.float32),
        ]
        + [pltpu.SemaphoreType.DMA] * NBUF,
    )
    def k(hs_hbm, src_hbm, dst_hbm, out_hbm, src_v, dst_v, rows_v, acc_sh, *gsem):
        cid = lax.axis_index("c")
        sid = lax.axis_index("s")
        wid = sid * NC + cid
        pltpu.sync_copy(src_hbm.at[wid], src_v)
        pltpu.sync_copy(dst_hbm.at[wid], dst_v)

        zeros = jnp.zeros((L,), jnp.float32)

        @pl.loop(0, CHUNK)
        def _(r):
            @pl.loop(0, HID_CH, step=L)
            def _(c):
                rows_v[0, r, pl.ds(c, L)] = zeros

        @pl.loop(0, RPT, step=CHUNK)
        def _(r0):
            pltpu.sync_copy(rows_v.at[0], acc_sh.at[pl.ds(sid * RPT + r0, CHUNK)])

        plsc.subcore_barrier()

        for b in range(NBUF):
            pltpu.async_copy(hs_hbm.at[src_v.at[b]], rows_v.at[b], gsem[b])

        @pl.loop(0, PT_CH, step=NBUF)
        def _(c):
            for b in range(NBUF):
                idx = c + b
                pltpu.make_async_copy(
                    hs_hbm.at[src_v.at[idx]], rows_v.at[b], gsem[b]
                ).wait()
                pltpu.sync_copy(rows_v.at[b], acc_sh.at[dst_v.at[idx]], add=True)

                @pl.when(idx + NBUF < PT_CH)
                def _():
                    pltpu.async_copy(
                        hs_hbm.at[src_v.at[idx + NBUF]], rows_v.at[b], gsem[b]
                    )

        plsc.subcore_barrier()

        pltpu.sync_copy(
            acc_sh.at[pl.ds(sid * RPT, RPT)],
            out_hbm.at[cid, pl.ds(sid * RPT, RPT)],
        )

    return k(hs, src3, dst3)


def _agg2_partials(gs, src3, dst3):
    @functools.partial(
        pl.kernel,
        out_type=jax.ShapeDtypeStruct((NW, PAD_N), jnp.float32),
        mesh=_mesh(),
        compiler_params=_sc_params(),
        scratch_types=[
            pltpu.VMEM((PT_CH, CHUNK), jnp.int32),
            pltpu.VMEM((PT_CH, CHUNK), jnp.int32),
            pltpu.VMEM((PAD_N,), jnp.float32),
            pltpu.VMEM((PAD_N,), jnp.float32),
        ],
    )
    def k(gs_hbm, src_hbm, dst_hbm, out_hbm, src_v, dst_v, tab_v, acc_v):
        wid = lax.axis_index("s") * NC + lax.axis_index("c")
        pltpu.sync_copy(src_hbm.at[wid], src_v)
        pltpu.sync_copy(dst_hbm.at[wid], dst_v)
        pltpu.sync_copy(gs_hbm, tab_v)
        zeros = jnp.zeros((L,), jnp.float32)

        @pl.loop(0, PAD_N, step=L)
        def _(i):
            acc_v[pl.ds(i, L)] = zeros

        @pl.loop(0, PT_CH)
        def _(c):
            @pl.loop(0, CHUNK, step=L)
            def _(j):
                si = src_v[c, pl.ds(j, L)]
                di = dst_v[c, pl.ds(j, L)]
                vals = plsc.load_gather(tab_v, [si])
                plsc.addupdate_scatter(acc_v, [di], vals)

        pltpu.sync_copy(acc_v, out_hbm.at[wid])

    return k(gs, src3, dst3)


def _tc_scale_matmul(deg_part, x_pad, W1):
    def body(dp_ref, x_ref, w1_ref, hs_ref, dis_ref):
        i = pl.program_id(0)
        deg = jnp.sum(dp_ref[...], axis=0)[:, None] + 1.0
        disv = lax.rsqrt(deg)
        rid = i * RB + lax.broadcasted_iota(jnp.int32, (RB, 1), 0)
        disv = jnp.where(rid < N_NODES, disv, 0.0)
        dis_ref[...] = disv
        h = jnp.dot(x_ref[...], w1_ref[...], preferred_element_type=jnp.float32)
        hs_ref[...] = h * disv

    return pl.pallas_call(
        body,
        grid=(PAD_N // RB,),
        in_specs=[
            pl.BlockSpec((NW, RB), lambda i: (0, i)),
            pl.BlockSpec((RB, IN_CH), lambda i: (i, 0)),
            pl.BlockSpec((IN_CH, HID_CH), lambda i: (0, 0)),
        ],
        out_specs=[
            pl.BlockSpec((RB, HID_CH), lambda i: (i, 0)),
            pl.BlockSpec((RB, 1), lambda i: (i, 0)),
        ],
        out_shape=[
            jax.ShapeDtypeStruct((PAD_N, HID_CH), jnp.float32),
            jax.ShapeDtypeStruct((PAD_N, 1), jnp.float32),
        ],
    )(deg_part, x_pad, W1)


def _tc_layer2_scalar(acc1, hs, dis, b1_row, w2_row):
    def body(acc_ref, hs_ref, dis_ref, b1_ref, w2_ref, gs_ref):
        s = acc_ref[0] + acc_ref[1] + hs_ref[...]
        pre = s * dis_ref[...] + b1_ref[...]
        h2 = jnp.maximum(pre, 0.0)
        g = jnp.sum(h2 * w2_ref[...], axis=1, keepdims=True)
        gs_ref[...] = g * dis_ref[...]

    return pl.pallas_call(
        body,
        grid=(PAD_N // RB,),
        in_specs=[
            pl.BlockSpec((NC, RB, HID_CH), lambda i: (0, i, 0)),
            pl.BlockSpec((RB, HID_CH), lambda i: (i, 0)),
            pl.BlockSpec((RB, 1), lambda i: (i, 0)),
            pl.BlockSpec((1, HID_CH), lambda i: (0, 0)),
            pl.BlockSpec((1, HID_CH), lambda i: (0, 0)),
        ],
        out_specs=pl.BlockSpec((RB, 1), lambda i: (i, 0)),
        out_shape=jax.ShapeDtypeStruct((PAD_N, 1), jnp.float32),
    )(acc1, hs, dis, b1_row, w2_row)


def _tc_final(acc2, gs, dis, b2_11):
    def body(a2_ref, gs_ref, dis_ref, b2_ref, o_ref):
        a2 = jnp.sum(a2_ref[...], axis=0)[:, None]
        o_ref[...] = jax.nn.sigmoid(
            (a2 + gs_ref[...]) * dis_ref[...] + b2_ref[...]
        )

    return pl.pallas_call(
        body,
        grid=(PAD_N // CB,),
        in_specs=[
            pl.BlockSpec((NW, CB), lambda i: (0, i)),
            pl.BlockSpec((CB, 1), lambda i: (i, 0)),
            pl.BlockSpec((CB, 1), lambda i: (i, 0)),
            pl.BlockSpec((1, 1), lambda i: (0, 0)),
        ],
        out_specs=pl.BlockSpec((CB, 1), lambda i: (i, 0)),
        out_shape=jax.ShapeDtypeStruct((PAD_N, 1), jnp.float32),
    )(acc2, gs, dis, b2_11)


def kernel(x, edge_index, W1, b1, W2, b2):
    ei = edge_index.astype(jnp.int32)
    pad = N_NODES + jnp.arange(E_PAD - N_EDGES, dtype=jnp.int32) % (PAD_N - N_NODES)
    src_flat = jnp.concatenate([ei[0], pad])
    dst_flat = jnp.concatenate([ei[1], pad])
    src3 = src_flat.reshape(NW, PT_CH, CHUNK)
    dst3 = dst_flat.reshape(NW, PT_CH, CHUNK)
    x_pad = jnp.pad(x, ((0, PAD_N - N_NODES), (0, 0)))

    deg_part = _deg_partials(dst3)
    hs, dis = _tc_scale_matmul(deg_part, x_pad, W1)
    acc1 = _agg1_partials(hs, src3, dst3)
    gs = _tc_layer2_scalar(
        acc1, hs, dis, b1.reshape(1, HID_CH), W2.reshape(1, HID_CH)
    )
    acc2 = _agg2_partials(gs.reshape(PAD_N), src3, dst3)
    out = _tc_final(acc2, gs, dis, b2.reshape(1, 1))
    return out[:N_NODES]


# speedup vs baseline: 3.3705x; 1.2581x over previous
import dataclasses
import functools

import jax
import jax.numpy as jnp
from jax import lax
from jax.experimental import pallas as pl
from jax.experimental.pallas import tpu as pltpu
from jax.experimental.pallas import tpu_sc as plsc

N_NODES = 10000
N_EDGES = 320000
IN_CH = 128
HID_CH = 64

NC = 2
NS = 16
NW = NC * NS
L = 16

E_T = N_EDGES // NW
CHUNK = 128
FT = E_T // CHUNK
TAIL = E_T - FT * CHUNK
NBUF = 6
PAD_N = 10240
NR = PAD_N // 128
RPT = PAD_N // NS
RB = 1024
CB = 2048


def _mesh():
    return plsc.VectorSubcoreMesh(core_axis_name="c", subcore_axis_name="s")


def _sc_params():
    cp = pltpu.CompilerParams()
    fields = pltpu.CompilerParams.__dataclass_fields__
    if "needs_layout_passes" in fields:
        cp = dataclasses.replace(cp, needs_layout_passes=False)
    if "use_tc_tiling_on_sc" in fields:
        cp = dataclasses.replace(cp, use_tc_tiling_on_sc=False)
    return cp


def _deg_partials(ei):
    @functools.partial(
        pl.kernel,
        out_type=jax.ShapeDtypeStruct((NW, PAD_N), jnp.float32),
        mesh=_mesh(),
        compiler_params=_sc_params(),
        scratch_types=[
            pltpu.VMEM((E_T,), jnp.int32),
            pltpu.VMEM((PAD_N,), jnp.float32),
        ],
    )
    def k(ei_hbm, out_hbm, dst_v, acc_v):
        wid = lax.axis_index("s") * NC + lax.axis_index("c")
        pltpu.sync_copy(ei_hbm.at[1, pl.ds(wid * E_T, E_T)], dst_v)
        zeros = jnp.zeros((L,), jnp.float32)
        ones = jnp.ones((L,), jnp.float32)

        @pl.loop(0, PAD_N, step=L)
        def _(i):
            acc_v[pl.ds(i, L)] = zeros

        @pl.loop(0, E_T, step=L)
        def _(j):
            idx = dst_v[pl.ds(j, L)]
            plsc.addupdate_scatter(acc_v, [idx], ones)

        pltpu.sync_copy(acc_v, out_hbm.at[wid])

    return k(ei)


def _agg1_partials(hs, ei):
    @functools.partial(
        pl.kernel,
        out_type=jax.ShapeDtypeStruct((NC, PAD_N, HID_CH), jnp.float32),
        mesh=_mesh(),
        compiler_params=_sc_params(),
        scratch_types=[
            pltpu.VMEM((E_T,), jnp.int32),
            pltpu.VMEM((E_T,), jnp.int32),
            pltpu.VMEM((NBUF, CHUNK, HID_CH), jnp.float32),
            pltpu.VMEM_SHARED((PAD_N, HID_CH), jnp.float32),
        ]
        + [pltpu.SemaphoreType.DMA] * NBUF,
    )
    def k(hs_hbm, ei_hbm, out_hbm, src_v, dst_v, rows_v, acc_sh, *gsem):
        cid = lax.axis_index("c")
        sid = lax.axis_index("s")
        wid = sid * NC + cid
        pltpu.sync_copy(ei_hbm.at[0, pl.ds(wid * E_T, E_T)], src_v)
        pltpu.sync_copy(ei_hbm.at[1, pl.ds(wid * E_T, E_T)], dst_v)

        zeros = jnp.zeros((L,), jnp.float32)

        @pl.loop(0, CHUNK)
        def _(r):
            @pl.loop(0, HID_CH, step=L)
            def _(c):
                rows_v[0, r, pl.ds(c, L)] = zeros

        @pl.loop(0, RPT, step=CHUNK)
        def _(r0):
            pltpu.sync_copy(rows_v.at[0], acc_sh.at[pl.ds(sid * RPT + r0, CHUNK)])

        plsc.subcore_barrier()

        for b in range(NBUF):
            pltpu.async_copy(
                hs_hbm.at[src_v.at[pl.ds(b * CHUNK, CHUNK)]], rows_v.at[b], gsem[b]
            )

        @pl.loop(0, FT, step=NBUF)
        def _(c):
            for b in range(NBUF):
                idx = c + b
                pltpu.make_async_copy(
                    hs_hbm.at[src_v.at[pl.ds(idx * CHUNK, CHUNK)]],
                    rows_v.at[b],
                    gsem[b],
                ).wait()
                pltpu.sync_copy(
                    rows_v.at[b],
                    acc_sh.at[dst_v.at[pl.ds(idx * CHUNK, CHUNK)]],
                    add=True,
                )

                @pl.when(idx + NBUF < FT)
                def _():
                    pltpu.async_copy(
                        hs_hbm.at[src_v.at[pl.ds((idx + NBUF) * CHUNK, CHUNK)]],
                        rows_v.at[b],
                        gsem[b],
                    )

        tail_rows = rows_v.at[0, pl.ds(0, TAIL)]
        pltpu.async_copy(
            hs_hbm.at[src_v.at[pl.ds(FT * CHUNK, TAIL)]], tail_rows, gsem[0]
        ).wait()
        pltpu.sync_copy(
            tail_rows, acc_sh.at[dst_v.at[pl.ds(FT * CHUNK, TAIL)]], add=True
        )

        plsc.subcore_barrier()

        pltpu.sync_copy(
            acc_sh.at[pl.ds(sid * RPT, RPT)],
            out_hbm.at[cid, pl.ds(sid * RPT, RPT)],
        )

    return k(hs, ei)


def _agg2_partials(gs2, ei):
    @functools.partial(
        pl.kernel,
        out_type=jax.ShapeDtypeStruct((NW, PAD_N), jnp.float32),
        mesh=_mesh(),
        compiler_params=_sc_params(),
        scratch_types=[
            pltpu.VMEM((E_T,), jnp.int32),
            pltpu.VMEM((E_T,), jnp.int32),
            pltpu.VMEM((NR, 128), jnp.float32),
            pltpu.VMEM((PAD_N,), jnp.float32),
        ],
    )
    def k(gs_hbm, ei_hbm, out_hbm, src_v, dst_v, tab_v, acc_v):
        wid = lax.axis_index("s") * NC + lax.axis_index("c")
        pltpu.sync_copy(ei_hbm.at[0, pl.ds(wid * E_T, E_T)], src_v)
        pltpu.sync_copy(ei_hbm.at[1, pl.ds(wid * E_T, E_T)], dst_v)
        pltpu.sync_copy(gs_hbm, tab_v)
        zeros = jnp.zeros((L,), jnp.float32)
        low = jnp.full((L,), 127, jnp.int32)

        @pl.loop(0, PAD_N, step=L)
        def _(i):
            acc_v[pl.ds(i, L)] = zeros

        @pl.loop(0, E_T, step=L)
        def _(j):
            si = src_v[pl.ds(j, L)]
            di = dst_v[pl.ds(j, L)]
            row = lax.shift_right_logical(si, 7)
            col = lax.bitwise_and(si, low)
            vals = plsc.load_gather(tab_v, [row, col])
            plsc.addupdate_scatter(acc_v, [di], vals)

        pltpu.sync_copy(acc_v, out_hbm.at[wid])

    return k(gs2, ei)


def _tc_scale_matmul(deg_part, x_pad, W1):
    def body(dp_ref, x_ref, w1_ref, hs_ref, dis_ref):
        i = pl.program_id(0)
        deg = jnp.sum(dp_ref[...], axis=0)[:, None] + 1.0
        disv = lax.rsqrt(deg)
        rid = i * RB + lax.broadcasted_iota(jnp.int32, (RB, 1), 0)
        disv = jnp.where(rid < N_NODES, disv, 0.0)
        dis_ref[...] = disv.reshape(RB // 128, 128)
        h = jnp.dot(x_ref[...], w1_ref[...], preferred_element_type=jnp.float32)
        hs_ref[...] = h * disv

    return pl.pallas_call(
        body,
        grid=(PAD_N // RB,),
        in_specs=[
            pl.BlockSpec((NW, RB), lambda i: (0, i)),
            pl.BlockSpec((RB, IN_CH), lambda i: (i, 0)),
            pl.BlockSpec((IN_CH, HID_CH), lambda i: (0, 0)),
        ],
        out_specs=[
            pl.BlockSpec((RB, HID_CH), lambda i: (i, 0)),
            pl.BlockSpec((RB // 128, 128), lambda i: (i, 0)),
        ],
        out_shape=[
            jax.ShapeDtypeStruct((PAD_N, HID_CH), jnp.float32),
            jax.ShapeDtypeStruct((NR, 128), jnp.float32),
        ],
    )(deg_part, x_pad, W1)


def _tc_layer2_scalar(acc1, hs, deg_part, b1_row, w2_row):
    def body(acc_ref, hs_ref, dp_ref, b1_ref, w2_ref, gs_ref):
        i = pl.program_id(0)
        deg = jnp.sum(dp_ref[...], axis=0)[:, None] + 1.0
        disv = lax.rsqrt(deg)
        rid = i * RB + lax.broadcasted_iota(jnp.int32, (RB, 1), 0)
        disv = jnp.where(rid < N_NODES, disv, 0.0)
        s = acc_ref[0] + acc_ref[1] + hs_ref[...]
        pre = s * disv + b1_ref[...]
        h2 = jnp.maximum(pre, 0.0)
        g = jnp.sum(h2 * w2_ref[...], axis=1, keepdims=True)
        gs_ref[...] = (g * disv).reshape(RB // 128, 128)

    return pl.pallas_call(
        body,
        grid=(PAD_N // RB,),
        in_specs=[
            pl.BlockSpec((NC, RB, HID_CH), lambda i: (0, i, 0)),
            pl.BlockSpec((RB, HID_CH), lambda i: (i, 0)),
            pl.BlockSpec((NW, RB), lambda i: (0, i)),
            pl.BlockSpec((1, HID_CH), lambda i: (0, 0)),
            pl.BlockSpec((1, HID_CH), lambda i: (0, 0)),
        ],
        out_specs=pl.BlockSpec((RB // 128, 128), lambda i: (i, 0)),
        out_shape=jax.ShapeDtypeStruct((NR, 128), jnp.float32),
    )(acc1, hs, deg_part, b1_row, w2_row)


def _tc_final(acc2, gs2, dis2, b2_11):
    def body(a2_ref, gs_ref, dis_ref, b2_ref, o_ref):
        a2 = jnp.sum(a2_ref[...], axis=0).reshape(CB // 128, 128)
        o_ref[...] = jax.nn.sigmoid(
            (a2 + gs_ref[...]) * dis_ref[...] + b2_ref[...]
        )

    return pl.pallas_call(
        body,
        grid=(PAD_N // CB,),
        in_specs=[
            pl.BlockSpec((NW, CB), lambda i: (0, i)),
            pl.BlockSpec((CB // 128, 128), lambda i: (i, 0)),
            pl.BlockSpec((CB // 128, 128), lambda i: (i, 0)),
            pl.BlockSpec((1, 1), lambda i: (0, 0)),
        ],
        out_specs=pl.BlockSpec((CB // 128, 128), lambda i: (i, 0)),
        out_shape=jax.ShapeDtypeStruct((NR, 128), jnp.float32),
    )(acc2, gs2, dis2, b2_11)


def kernel(x, edge_index, W1, b1, W2, b2):
    ei = edge_index.astype(jnp.int32)
    x_pad = jnp.pad(x, ((0, PAD_N - N_NODES), (0, 0)))

    deg_part = _deg_partials(ei)
    hs, dis2 = _tc_scale_matmul(deg_part, x_pad, W1)
    acc1 = _agg1_partials(hs, ei)
    gs2 = _tc_layer2_scalar(
        acc1, hs, deg_part, b1.reshape(1, HID_CH), W2.reshape(1, HID_CH)
    )
    acc2 = _agg2_partials(gs2, ei)
    out = _tc_final(acc2, gs2, dis2, b2.reshape(1, 1))
    return out.reshape(PAD_N, 1)[:N_NODES]
